# Initial kernel scaffold; baseline (speedup 1.0000x reference)
import functools

import jax
import jax.numpy as jnp
from jax import lax
from jax.experimental import pallas as pl
from jax.experimental.pallas import tpu as pltpu
from jax.experimental.pallas import tpu_sc as plsc

N = 10000
D = 128
E = 320000
LANES = 16

NC = 2
NS = 16
NW = NC * NS
EPT = E // NW
K = 80
CHUNKS = EPT // K
RPT = N // NS
ZR = 125


def _sc_agg(x, src, dst, ea):
    mesh = plsc.VectorSubcoreMesh(core_axis_name="c", subcore_axis_name="s")

    @functools.partial(
        pl.kernel,
        out_type=jax.ShapeDtypeStruct((NC, N, D), jnp.float32),
        mesh=mesh,
        scratch_types=[
            pltpu.VMEM((K,), jnp.int32),
            pltpu.VMEM((K,), jnp.int32),
            pltpu.VMEM((K,), jnp.float32),
            pltpu.VMEM((K, D), jnp.float32),
            pltpu.VMEM((ZR, D), jnp.float32),
            pltpu.VMEM_SHARED((N, D), jnp.float32),
            pltpu.SemaphoreType.DMA,
        ],
    )
    def k(x_hbm, src_hbm, dst_hbm, ea_hbm, out_hbm,
          src_v, dst_v, ea_v, rows_v, zero_v, acc_sh, sem):
        cid = lax.axis_index("c")
        sid = lax.axis_index("s")
        wid = cid * NS + sid

        def zrow(i, carry):
            for r in range(D // LANES):
                zero_v[i, pl.ds(r * LANES, LANES)] = jnp.zeros(
                    (LANES,), jnp.float32)
            return carry
        lax.fori_loop(0, ZR, zrow, 0)
        for t in range(RPT // ZR):
            pltpu.sync_copy(zero_v,
                            acc_sh.at[pl.ds(sid * RPT + t * ZR, ZR)])
        plsc.subcore_barrier()

        def chunk(c, carry):
            base = pl.multiple_of(wid * EPT + c * K, K)
            pltpu.sync_copy(src_hbm.at[pl.ds(base, K)], src_v)
            pltpu.sync_copy(dst_hbm.at[pl.ds(base, K)], dst_v)
            pltpu.sync_copy(ea_hbm.at[pl.ds(base, K)], ea_v)
            pltpu.async_copy(x_hbm.at[src_v], rows_v, sem).wait()

            def scale(j, c2):
                a = ea_v[j]
                for r in range(D // LANES):
                    sl = pl.ds(r * LANES, LANES)
                    rows_v[j, sl] = rows_v[j, sl] * a
                return c2
            lax.fori_loop(0, K, scale, 0)
            pltpu.sync_copy(rows_v, acc_sh.at[dst_v], add=True)
            return carry
        lax.fori_loop(0, CHUNKS, chunk, 0)
        plsc.subcore_barrier()

        pltpu.sync_copy(acc_sh.at[pl.ds(sid * RPT, RPT)],
                        out_hbm.at[cid, pl.ds(sid * RPT, RPT)])

    return k(x, src, dst, ea)


_BN = 400


def _mm_body_relu(p_ref, w_ref, b_ref, o_ref):
    a = p_ref[0] + p_ref[1]
    h = jnp.dot(a, w_ref[...], preferred_element_type=jnp.float32)
    o_ref[...] = jnp.maximum(h + b_ref[...], 0.0)


def _mm_body_lin(p_ref, w_ref, b_ref, o_ref):
    a = p_ref[0] + p_ref[1]
    h = jnp.dot(a, w_ref[...], preferred_element_type=jnp.float32)
    o_ref[...] = h + b_ref[...]


def _mm(p, w, b, relu):
    body = _mm_body_relu if relu else _mm_body_lin
    return pl.pallas_call(
        body,
        grid=(N // _BN,),
        in_specs=[
            pl.BlockSpec((NC, _BN, D), lambda i: (0, i, 0)),
            pl.BlockSpec((D, D), lambda i: (0, 0)),
            pl.BlockSpec((1, D), lambda i: (0, 0)),
        ],
        out_specs=pl.BlockSpec((_BN, D), lambda i: (i, 0)),
        out_shape=jax.ShapeDtypeStruct((N, D), jnp.float32),
    )(p, w, b.reshape(1, D))


def kernel(x, edge_index, edge_attr, W1, b1, W2, b2):
    src = edge_index[0]
    dst = edge_index[1]
    p1 = _sc_agg(x, src, dst, edge_attr)
    h = _mm(p1, W1, b1, relu=True)
    p2 = _sc_agg(h, src, dst, edge_attr)
    return _mm(p2, W2, b2, relu=False)


# speedup vs baseline: 3.8169x; 3.8169x over previous
import functools

import jax
import jax.numpy as jnp
from jax import lax
from jax.experimental import pallas as pl
from jax.experimental.pallas import tpu as pltpu
from jax.experimental.pallas import tpu_sc as plsc

N = 10000
D = 128
E = 320000
LANES = 16

NC = 2
NS = 16
NW = NC * NS
EPT = E // NW
K = 80
CHUNKS = EPT // K
NP = 10240
RPT = NP // NS
ZR = 128


def _sc_agg(x, src, dst, ea):
    mesh = plsc.VectorSubcoreMesh(core_axis_name="c", subcore_axis_name="s")

    @functools.partial(
        pl.kernel,
        out_type=jax.ShapeDtypeStruct((NC, NS, RPT, D), jnp.float32),
        mesh=mesh,
        scratch_types=[
            pltpu.VMEM((K,), jnp.int32),
            pltpu.VMEM((K,), jnp.int32),
            pltpu.VMEM((K,), jnp.float32),
            pltpu.VMEM((K, D), jnp.float32),
            pltpu.VMEM((ZR, D), jnp.float32),
            pltpu.VMEM_SHARED((NP, D), jnp.float32),
            pltpu.SemaphoreType.DMA,
        ],
    )
    def k(x_hbm, src_hbm, dst_hbm, ea_hbm, out_hbm,
          src_v, dst_v, ea_v, rows_v, zero_v, acc_sh, sem):
        cid = lax.axis_index("c")
        sid = lax.axis_index("s")
        wid = cid * NS + sid

        def zrow(i, carry):
            for r in range(D // LANES):
                zero_v[i, pl.ds(r * LANES, LANES)] = jnp.zeros(
                    (LANES,), jnp.float32)
            return carry
        lax.fori_loop(0, ZR, zrow, 0)
        for t in range(RPT // ZR):
            pltpu.sync_copy(zero_v,
                            acc_sh.at[pl.ds(sid * RPT + t * ZR, ZR)])
        plsc.subcore_barrier()

        def chunk(c, carry):
            base = pl.multiple_of(wid * EPT + c * K, K)
            pltpu.sync_copy(src_hbm.at[pl.ds(base, K)], src_v)
            pltpu.sync_copy(dst_hbm.at[pl.ds(base, K)], dst_v)
            pltpu.sync_copy(ea_hbm.at[pl.ds(base, K)], ea_v)
            pltpu.async_copy(x_hbm.at[src_v], rows_v, sem).wait()

            def scale(g, c2):
                eav = ea_v[pl.ds(g * LANES, LANES)]
                for i in range(LANES):
                    a = eav[i]
                    j = g * LANES + i
                    for r in range(D // LANES):
                        sl = pl.ds(r * LANES, LANES)
                        rows_v[j, sl] = rows_v[j, sl] * a
                return c2
            lax.fori_loop(0, K // LANES, scale, 0)
            pltpu.sync_copy(rows_v, acc_sh.at[dst_v], add=True)
            return carry
        lax.fori_loop(0, CHUNKS, chunk, 0)
        plsc.subcore_barrier()

        pltpu.sync_copy(acc_sh.at[pl.ds(sid * RPT, RPT)],
                        out_hbm.at[cid, sid])

    return k(x, src, dst, ea).reshape(NC, NP, D)


_BN = 400


def _mm_body_relu(p_ref, w_ref, b_ref, o_ref):
    a = p_ref[0] + p_ref[1]
    h = jnp.dot(a, w_ref[...], preferred_element_type=jnp.float32)
    o_ref[...] = jnp.maximum(h + b_ref[...], 0.0)


def _mm_body_lin(p_ref, w_ref, b_ref, o_ref):
    a = p_ref[0] + p_ref[1]
    h = jnp.dot(a, w_ref[...], preferred_element_type=jnp.float32)
    o_ref[...] = h + b_ref[...]


def _mm(p, w, b, relu):
    body = _mm_body_relu if relu else _mm_body_lin
    return pl.pallas_call(
        body,
        grid=(N // _BN,),
        in_specs=[
            pl.BlockSpec((NC, _BN, D), lambda i: (0, i, 0)),
            pl.BlockSpec((D, D), lambda i: (0, 0)),
            pl.BlockSpec((1, D), lambda i: (0, 0)),
        ],
        out_specs=pl.BlockSpec((_BN, D), lambda i: (i, 0)),
        out_shape=jax.ShapeDtypeStruct((N, D), jnp.float32),
    )(p, w, b.reshape(1, D))


def kernel(x, edge_index, edge_attr, W1, b1, W2, b2):
    src = edge_index[0]
    dst = edge_index[1]
    p1 = _sc_agg(x, src, dst, edge_attr)
    h = _mm(p1, W1, b1, relu=True)
    p2 = _sc_agg(h, src, dst, edge_attr)
    return _mm(p2, W2, b2, relu=False)
